# speedup vs baseline: 2.4599x; 2.4599x over previous
import functools

import jax
import jax.numpy as jnp
from jax import lax
from jax.experimental import pallas as pl
from jax.experimental.pallas import tpu as pltpu
from jax.experimental.pallas import tpu_sc as plsc

B = 4
S = 8192
D = 768
N = B * S
NC = 2
NS = 16
NW = NC * NS
PER_W = N // NW
CH = 64
NCHUNK = PER_W // CH
NBUF = 2

_mesh = plsc.VectorSubcoreMesh(core_axis_name="c", subcore_axis_name="s")


@functools.partial(
    pl.kernel,
    mesh=_mesh,
    out_type=jax.ShapeDtypeStruct((N, D), jnp.float32),
    scratch_types=[
        pltpu.VMEM((PER_W,), jnp.int32),
        pltpu.VMEM((NBUF, CH, D), jnp.float32),
        pltpu.SemaphoreType.DMA,
        pltpu.SemaphoreType.DMA,
        pltpu.SemaphoreType.DMA,
        pltpu.SemaphoreType.DMA,
    ],
)
def _gather_rows(idx_hbm, table_hbm, out_hbm, idx_v, rows_v,
                 gsem0, gsem1, ssem0, ssem1):
    gsems = (gsem0, gsem1)
    ssems = (ssem0, ssem1)
    wid = lax.axis_index("s") * NC + lax.axis_index("c")
    base = wid * PER_W

    pltpu.sync_copy(idx_hbm.at[pl.ds(base, PER_W)], idx_v)

    gathers = [None] * NCHUNK
    for c in range(NBUF):
        gathers[c] = pltpu.async_copy(
            table_hbm.at[idx_v.at[pl.ds(c * CH, CH)]],
            rows_v.at[c],
            gsems[c],
        )

    tail = []
    for c in range(NCHUNK):
        b = c % NBUF
        gathers[c].wait()
        scat = pltpu.async_copy(
            rows_v.at[b],
            out_hbm.at[pl.ds(base + c * CH, CH)],
            ssems[b],
        )
        nxt = c + NBUF
        if nxt < NCHUNK:
            scat.wait()
            gathers[nxt] = pltpu.async_copy(
                table_hbm.at[idx_v.at[pl.ds(nxt * CH, CH)]],
                rows_v.at[b],
                gsems[b],
            )
        else:
            tail.append(scat)
    for scat in tail:
        scat.wait()


def kernel(src_seq, pos_table):
    idx = src_seq.astype(jnp.int32).reshape(N)
    out = _gather_rows(idx, pos_table)
    return out.reshape(B, S, D)


# speedup vs baseline: 2.4776x; 1.0072x over previous
import functools

import jax
import jax.numpy as jnp
from jax import lax
from jax.experimental import pallas as pl
from jax.experimental.pallas import tpu as pltpu
from jax.experimental.pallas import tpu_sc as plsc

B = 4
S = 8192
D = 768
N = B * S
NC = 2
NS = 16
NW = NC * NS
PER_W = N // NW
CH = 32
NCHUNK = PER_W // CH
NBUF = 4

_mesh = plsc.VectorSubcoreMesh(core_axis_name="c", subcore_axis_name="s")


@functools.partial(
    pl.kernel,
    mesh=_mesh,
    out_type=jax.ShapeDtypeStruct((N, D), jnp.float32),
    scratch_types=[
        pltpu.VMEM((PER_W,), jnp.int32),
        pltpu.VMEM((NBUF, CH, D), jnp.float32),
    ] + [pltpu.SemaphoreType.DMA] * (2 * NBUF),
)
def _gather_rows(idx_hbm, table_hbm, out_hbm, idx_v, rows_v, *sems):
    gsems = sems[:NBUF]
    ssems = sems[NBUF:]
    wid = lax.axis_index("s") * NC + lax.axis_index("c")
    base = wid * PER_W

    pltpu.sync_copy(idx_hbm.at[pl.ds(base, PER_W)], idx_v)

    gathers = [None] * NCHUNK
    for c in range(NBUF):
        gathers[c] = pltpu.async_copy(
            table_hbm.at[idx_v.at[pl.ds(c * CH, CH)]],
            rows_v.at[c],
            gsems[c],
        )

    tail = []
    for c in range(NCHUNK):
        b = c % NBUF
        gathers[c].wait()
        scat = pltpu.async_copy(
            rows_v.at[b],
            out_hbm.at[pl.ds(base + c * CH, CH)],
            ssems[b],
        )
        nxt = c + NBUF
        if nxt < NCHUNK:
            scat.wait()
            gathers[nxt] = pltpu.async_copy(
                table_hbm.at[idx_v.at[pl.ds(nxt * CH, CH)]],
                rows_v.at[b],
                gsems[b],
            )
        else:
            tail.append(scat)
    for scat in tail:
        scat.wait()


def kernel(src_seq, pos_table):
    idx = src_seq.astype(jnp.int32).reshape(N)
    out = _gather_rows(idx, pos_table)
    return out.reshape(B, S, D)
